# baseline (speedup 1.0000x reference)
import functools

import jax
import jax.numpy as jnp
from jax import lax
from jax.experimental import pallas as pl
from jax.experimental.pallas import tpu as pltpu
from jax.experimental.pallas import tpu_sc as plsc

N = 10000
E = 320000
D = 128
H = 48
K = 5

NC = 2
NS = 16
NW = NC * NS
EPW = E // NW
CH = 80
NCHUNK = EPW // CH
ROWS_PER_TILE = N // NS
DEG_PER_TILE = 640
NPAD = NS * DEG_PER_TILE


def _mesh():
    return plsc.VectorSubcoreMesh(core_axis_name="c", subcore_axis_name="s")


def _wid(c, s):
    return c * NS + s


def _deg_kernel(row_hbm, col_hbm, out_hbm, idx_r, idx_c, wbuf, acc_sh):
    c = lax.axis_index("c")
    s = lax.axis_index("s")
    wid = _wid(c, s)
    base = wid * EPW

    for j in range(CH // 16):
        wbuf[pl.ds(j * 16, 16)] = jnp.zeros((16,), jnp.float32)
    for j in range(DEG_PER_TILE // CH):
        pltpu.sync_copy(wbuf, acc_sh.at[pl.ds(s * DEG_PER_TILE + j * CH, CH)])
    plsc.subcore_barrier()

    def body(it, _):
        pltpu.sync_copy(row_hbm.at[pl.ds(base + it * CH, CH)], idx_r)
        pltpu.sync_copy(col_hbm.at[pl.ds(base + it * CH, CH)], idx_c)
        for j in range(CH // 16):
            r16 = idx_r[pl.ds(j * 16, 16)]
            c16 = idx_c[pl.ds(j * 16, 16)]
            w16 = jnp.where(r16 == c16, 0.0, 1.0).astype(jnp.float32)
            wbuf[pl.ds(j * 16, 16)] = w16
        pltpu.sync_copy(wbuf, acc_sh.at[idx_r], add=True)
        return ()

    lax.fori_loop(0, NCHUNK, body, (), unroll=False)
    plsc.subcore_barrier()
    pltpu.sync_copy(acc_sh.at[pl.ds(s * DEG_PER_TILE, DEG_PER_TILE)],
                    out_hbm.at[c, pl.ds(s * DEG_PER_TILE, DEG_PER_TILE)])


def _deg_partials(row, col):
    kfn = pl.kernel(
        _deg_kernel,
        out_type=jax.ShapeDtypeStruct((NC, NPAD), jnp.float32),
        mesh=_mesh(),
        compiler_params=pltpu.CompilerParams(needs_layout_passes=False),
        scratch_types=[
            pltpu.VMEM((CH,), jnp.int32),
            pltpu.VMEM((CH,), jnp.int32),
            pltpu.VMEM((CH,), jnp.float32),
            pltpu.VMEM_SHARED((NPAD,), jnp.float32),
        ],
    )
    return kfn(row, col)


def _dinv_body(p_ref, out_ref):
    deg = jnp.sum(p_ref[...], axis=0, keepdims=True)
    out_ref[...] = jnp.where(
        deg > 0.0, lax.rsqrt(jnp.maximum(deg, 1e-12)), 0.0)


def _tc_dinv(degp):
    return pl.pallas_call(
        _dinv_body,
        out_shape=jax.ShapeDtypeStruct((1, NPAD), jnp.float32),
    )(degp)


def _lapw_kernel(row_hbm, col_hbm, dinv_hbm, scale_hbm, out_hbm,
                 dinv_v, idx_r, idx_c, lw_v, scale_v):
    c = lax.axis_index("c")
    s = lax.axis_index("s")
    wid = _wid(c, s)
    base = wid * EPW

    pltpu.sync_copy(dinv_hbm.at[0], dinv_v)
    pltpu.sync_copy(scale_hbm, scale_v)
    scale16 = scale_v[...]

    def body(it, _):
        pltpu.sync_copy(row_hbm.at[pl.ds(base + it * CH, CH)], idx_r)
        pltpu.sync_copy(col_hbm.at[pl.ds(base + it * CH, CH)], idx_c)
        for j in range(CH // 16):
            r16 = idx_r[pl.ds(j * 16, 16)]
            c16 = idx_c[pl.ds(j * 16, 16)]
            dr = plsc.load_gather(dinv_v, [r16])
            dc = plsc.load_gather(dinv_v, [c16])
            w = jnp.where(r16 == c16, 0.0, 1.0).astype(jnp.float32)
            lw_v[pl.ds(j * 16, 16)] = scale16 * dr * dc * w
        pltpu.sync_copy(lw_v, out_hbm.at[pl.ds(base + it * CH, CH)])
        return ()

    lax.fori_loop(0, NCHUNK, body, (), unroll=False)


def _lap_weights(row, col, dinv, scale16):
    kfn = pl.kernel(
        _lapw_kernel,
        out_type=jax.ShapeDtypeStruct((E,), jnp.float32),
        mesh=_mesh(),
        compiler_params=pltpu.CompilerParams(needs_layout_passes=False),
        scratch_types=[
            pltpu.VMEM((NPAD,), jnp.float32),
            pltpu.VMEM((CH,), jnp.int32),
            pltpu.VMEM((CH,), jnp.int32),
            pltpu.VMEM((CH,), jnp.float32),
            pltpu.VMEM((16,), jnp.float32),
        ],
    )
    return kfn(row, col, dinv, scale16)


def _matvec_kernel(F, v_hbm, row3_hbm, col_hbm, lw_hbm, zeros_hbm, out_hbm,
                   idxr_all, idxc_all, lw_all, rows2, sem0, sem1,
                   semp0, semp1, acc_sh):
    c = lax.axis_index("c")
    s = lax.axis_index("s")
    wid = _wid(c, s)
    base = wid * EPW
    FG = F // 16

    h_r = pltpu.async_copy(row3_hbm.at[wid], idxr_all, semp0)
    h_c = pltpu.async_copy(col_hbm.at[pl.ds(base, EPW)], idxc_all, semp1)
    h_l = pltpu.async_copy(lw_hbm.at[pl.ds(base, EPW)], lw_all, sem1)

    @pl.when(s < NS - 1)
    def _():
        cp = pltpu.make_async_copy(zeros_hbm.at[pl.ds(0, 624)],
                                   acc_sh.at[pl.ds(s * 624, 624)], sem0)
        cp.start()
        cp.wait()

    @pl.when(s == NS - 1)
    def _():
        cp = pltpu.make_async_copy(zeros_hbm,
                                   acc_sh.at[pl.ds((NS - 1) * 624, 640)],
                                   sem0)
        cp.start()
        cp.wait()

    h_r.wait()
    h_c.wait()
    h_l.wait()
    plsc.subcore_barrier()

    def start_gather(ck, buf, sem):
        return pltpu.async_copy(
            v_hbm.at[idxc_all.at[pl.ds(ck * CH, CH)]], buf, sem)

    def drain(buf, sem):
        pltpu.make_async_copy(v_hbm.at[pl.ds(0, CH)], buf, sem).wait()

    def scale_scatter(ck, buf):
        @plsc.parallel_loop(0, CH // 8, 1, unroll=4)
        def _(i8):
            for d in range(8):
                e = i8 * 8 + d
                lwb = plsc.load_gather(
                    lw_all, [jnp.full((16,), ck * CH + e, jnp.int32)])
                for j in range(FG):
                    buf[e, pl.ds(j * 16, 16)] = buf[e, pl.ds(j * 16, 16)] * lwb
        pltpu.sync_copy(buf, acc_sh.at[idxr_all.at[ck]], add=True)

    start_gather(0, rows2.at[0], sem0)

    def pair_body(i, _):
        c0 = 2 * i
        h1 = start_gather(c0 + 1, rows2.at[1], sem1)
        drain(rows2.at[0], sem0)
        scale_scatter(c0, rows2.at[0])
        start_gather(c0 + 2, rows2.at[0], sem0)
        h1.wait()
        scale_scatter(c0 + 1, rows2.at[1])
        return ()

    lax.fori_loop(0, (NCHUNK - 1) // 2, pair_body, (), unroll=False)
    drain(rows2.at[0], sem0)
    scale_scatter(NCHUNK - 1, rows2.at[0])
    plsc.subcore_barrier()

    @pl.when(s < NS - 1)
    def _():
        pltpu.sync_copy(acc_sh.at[pl.ds(s * 624, 624)],
                        out_hbm.at[c, pl.ds(s * 624, 624)])

    @pl.when(s == NS - 1)
    def _():
        pltpu.sync_copy(acc_sh.at[pl.ds((NS - 1) * 624, 640)],
                        out_hbm.at[c, pl.ds((NS - 1) * 624, 640)])


def _sc_matvec(v, row3, col, lw, zeros_tile, F):
    kfn = pl.kernel(
        functools.partial(_matvec_kernel, F),
        out_type=jax.ShapeDtypeStruct((NC, N, F), jnp.float32),
        mesh=_mesh(),
        compiler_params=pltpu.CompilerParams(needs_layout_passes=False,
                                             use_tc_tiling_on_sc=False),
        scratch_types=[
            pltpu.VMEM((NCHUNK, CH), jnp.int32),
            pltpu.VMEM((EPW,), jnp.int32),
            pltpu.VMEM((EPW,), jnp.float32),
            pltpu.VMEM((2, CH, F), jnp.float32),
            pltpu.SemaphoreType.DMA,
            pltpu.SemaphoreType.DMA,
            pltpu.SemaphoreType.DMA,
            pltpu.SemaphoreType.DMA,
            pltpu.VMEM_SHARED((N, F), jnp.float32),
        ],
    )
    return kfn(v, row3, col, lw, zeros_tile)


BLK = 1000
GRID = N // BLK


def _step1_body(diag_ref, p_ref, v_ref, w0_ref, w1_ref, t1_ref, out_ref):
    diag = diag_ref[0, 0]
    v = v_ref[...]
    t1 = p_ref[0] + p_ref[1] + diag * v
    t1_ref[...] = t1
    out_ref[...] = (jnp.dot(v, w0_ref[...], preferred_element_type=jnp.float32)
                    + jnp.dot(t1, w1_ref[...], preferred_element_type=jnp.float32))


def _tc_step1(p, v, w0, w1, diag, F):
    return pl.pallas_call(
        _step1_body,
        grid=(GRID,),
        in_specs=[
            pl.BlockSpec((1, 1), lambda i: (0, 0), memory_space=pltpu.SMEM),
            pl.BlockSpec((NC, BLK, F), lambda i: (0, i, 0)),
            pl.BlockSpec((BLK, F), lambda i: (i, 0)),
            pl.BlockSpec((F, H), lambda i: (0, 0)),
            pl.BlockSpec((F, H), lambda i: (0, 0)),
        ],
        out_specs=[
            pl.BlockSpec((BLK, F), lambda i: (i, 0)),
            pl.BlockSpec((BLK, H), lambda i: (i, 0)),
        ],
        out_shape=[
            jax.ShapeDtypeStruct((N, F), jnp.float32),
            jax.ShapeDtypeStruct((N, H), jnp.float32),
        ],
    )(diag, p, v, w0, w1)


def _stepk_body(last, diag_ref, p_ref, v_ref, tp_ref, wk_ref, oprev_ref,
                b_ref, tn_ref, out_ref):
    diag = diag_ref[0, 0]
    tn = 2.0 * (p_ref[0] + p_ref[1] + diag * v_ref[...]) - tp_ref[...]
    tn_ref[...] = tn
    o = oprev_ref[...] + jnp.dot(tn, wk_ref[...],
                                 preferred_element_type=jnp.float32)
    if last:
        o = jnp.maximum(o + b_ref[...], 0.0)
    out_ref[...] = o


def _tc_stepk(p, v, tprev, wk, outprev, b, diag, F, last):
    return pl.pallas_call(
        functools.partial(_stepk_body, last),
        grid=(GRID,),
        in_specs=[
            pl.BlockSpec((1, 1), lambda i: (0, 0), memory_space=pltpu.SMEM),
            pl.BlockSpec((NC, BLK, F), lambda i: (0, i, 0)),
            pl.BlockSpec((BLK, F), lambda i: (i, 0)),
            pl.BlockSpec((BLK, F), lambda i: (i, 0)),
            pl.BlockSpec((F, H), lambda i: (0, 0)),
            pl.BlockSpec((BLK, H), lambda i: (i, 0)),
            pl.BlockSpec((1, H), lambda i: (0, 0)),
        ],
        out_specs=[
            pl.BlockSpec((BLK, F), lambda i: (i, 0)),
            pl.BlockSpec((BLK, H), lambda i: (i, 0)),
        ],
        out_shape=[
            jax.ShapeDtypeStruct((N, F), jnp.float32),
            jax.ShapeDtypeStruct((N, H), jnp.float32),
        ],
    )(diag, p, v, tprev, wk, outprev, b)


def _pool_body(h_ref, f1w_ref, f1b_ref, f2w_ref, f2b_ref, out_ref, acc_ref):
    i = pl.program_id(0)

    @pl.when(i == 0)
    def _():
        acc_ref[...] = jnp.zeros_like(acc_ref)

    acc_ref[...] += jnp.sum(h_ref[...], axis=0, keepdims=True)

    @pl.when(i == GRID - 1)
    def _():
        z = jnp.maximum(
            jnp.dot(acc_ref[...], f1w_ref[...],
                    preferred_element_type=jnp.float32) + f1b_ref[...], 0.0)
        out_ref[...] = (jnp.dot(z, f2w_ref[...],
                                preferred_element_type=jnp.float32)
                        + f2b_ref[...])


def _tc_pool(h, f1w, f1b, f2w, f2b):
    return pl.pallas_call(
        _pool_body,
        grid=(GRID,),
        in_specs=[
            pl.BlockSpec((BLK, H), lambda i: (i, 0)),
            pl.BlockSpec((H, 32), lambda i: (0, 0)),
            pl.BlockSpec((1, 32), lambda i: (0, 0)),
            pl.BlockSpec((32, 1), lambda i: (0, 0)),
            pl.BlockSpec((1, 1), lambda i: (0, 0)),
        ],
        out_specs=pl.BlockSpec((1, 1), lambda i: (0, 0)),
        out_shape=jax.ShapeDtypeStruct((1, 1), jnp.float32),
        scratch_shapes=[pltpu.VMEM((1, H), jnp.float32)],
    )(h, f1w, f1b, f2w, f2b)


def _cheb_layer(v, W, b, row3, col, lw, diag, zeros_tile):
    F = v.shape[1]
    p = _sc_matvec(v, row3, col, lw, zeros_tile, F)
    t1, out = _tc_step1(p, v, W[0], W[1], diag, F)
    t0 = v
    for k in range(2, K):
        p = _sc_matvec(t1, row3, col, lw, zeros_tile, F)
        tn, out = _tc_stepk(p, t1, t0, W[k], out, b, diag, F, last=(k == K - 1))
        t0, t1 = t1, tn
    return out


def kernel(x, edge_index, lmax, batch, W1, b1, W2, b2, W3, b3, W4, b4,
           W5, b5, fc1_W, fc1_b, fc2_W, fc2_b):
    row = edge_index[0]
    col = edge_index[1]
    lm = lmax[0]
    scale16 = jnp.full((16,), -2.0 / lm, jnp.float32)
    diag = jnp.reshape(2.0 / lm - 1.0, (1, 1)).astype(jnp.float32)

    degp = _deg_partials(row, col)
    dinv = _tc_dinv(degp)
    lw = _lap_weights(row, col, dinv, scale16)

    zeros_d = jnp.zeros((640, D), jnp.float32)
    zeros_h = jnp.zeros((640, H), jnp.float32)

    row3 = jnp.reshape(row, (NW, NCHUNK, CH))

    h = _cheb_layer(x, W1, jnp.reshape(b1, (1, H)), row3, col, lw, diag, zeros_d)
    for W, b in ((W2, b2), (W3, b3), (W4, b4), (W5, b5)):
        h = _cheb_layer(h, W, jnp.reshape(b, (1, H)), row3, col, lw, diag, zeros_h)

    return _tc_pool(h, fc1_W, jnp.reshape(fc1_b, (1, 32)), fc2_W,
                    jnp.reshape(fc2_b, (1, 1)))


# speedup vs baseline: 1.1137x; 1.1137x over previous
import functools

import jax
import jax.numpy as jnp
from jax import lax
from jax.experimental import pallas as pl
from jax.experimental.pallas import tpu as pltpu
from jax.experimental.pallas import tpu_sc as plsc

N = 10000
E = 320000
D = 128
H = 48
K = 5

NC = 2
NS = 16
NW = NC * NS
EPW = E // NW
CH = 80
NCHUNK = EPW // CH
ROWS_PER_TILE = N // NS
DEG_PER_TILE = 640
NPAD = NS * DEG_PER_TILE


def _mesh():
    return plsc.VectorSubcoreMesh(core_axis_name="c", subcore_axis_name="s")


def _wid(c, s):
    return c * NS + s


def _deg_kernel(row_hbm, col_hbm, out_hbm, idx_r, idx_c, wbuf, acc_sh):
    c = lax.axis_index("c")
    s = lax.axis_index("s")
    wid = _wid(c, s)
    base = wid * EPW

    for j in range(CH // 16):
        wbuf[pl.ds(j * 16, 16)] = jnp.zeros((16,), jnp.float32)
    for j in range(DEG_PER_TILE // CH):
        pltpu.sync_copy(wbuf, acc_sh.at[pl.ds(s * DEG_PER_TILE + j * CH, CH)])
    plsc.subcore_barrier()

    def body(it, _):
        pltpu.sync_copy(row_hbm.at[pl.ds(base + it * CH, CH)], idx_r)
        pltpu.sync_copy(col_hbm.at[pl.ds(base + it * CH, CH)], idx_c)
        for j in range(CH // 16):
            r16 = idx_r[pl.ds(j * 16, 16)]
            c16 = idx_c[pl.ds(j * 16, 16)]
            w16 = jnp.where(r16 == c16, 0.0, 1.0).astype(jnp.float32)
            wbuf[pl.ds(j * 16, 16)] = w16
        pltpu.sync_copy(wbuf, acc_sh.at[idx_r], add=True)
        return ()

    lax.fori_loop(0, NCHUNK, body, (), unroll=False)
    plsc.subcore_barrier()
    pltpu.sync_copy(acc_sh.at[pl.ds(s * DEG_PER_TILE, DEG_PER_TILE)],
                    out_hbm.at[c, pl.ds(s * DEG_PER_TILE, DEG_PER_TILE)])


def _deg_partials(row, col):
    kfn = pl.kernel(
        _deg_kernel,
        out_type=jax.ShapeDtypeStruct((NC, NPAD), jnp.float32),
        mesh=_mesh(),
        compiler_params=pltpu.CompilerParams(needs_layout_passes=False),
        scratch_types=[
            pltpu.VMEM((CH,), jnp.int32),
            pltpu.VMEM((CH,), jnp.int32),
            pltpu.VMEM((CH,), jnp.float32),
            pltpu.VMEM_SHARED((NPAD,), jnp.float32),
        ],
    )
    return kfn(row, col)


def _dinv_body(p_ref, out_ref):
    deg = jnp.sum(p_ref[...], axis=0, keepdims=True)
    out_ref[...] = jnp.where(
        deg > 0.0, lax.rsqrt(jnp.maximum(deg, 1e-12)), 0.0)


def _tc_dinv(degp):
    return pl.pallas_call(
        _dinv_body,
        out_shape=jax.ShapeDtypeStruct((1, NPAD), jnp.float32),
    )(degp)


def _lapw_kernel(row_hbm, col_hbm, dinv_hbm, scale_hbm, out_hbm,
                 dinv_v, idx_r, idx_c, lw_v, scale_v):
    c = lax.axis_index("c")
    s = lax.axis_index("s")
    wid = _wid(c, s)
    base = wid * EPW

    pltpu.sync_copy(dinv_hbm.at[0], dinv_v)
    pltpu.sync_copy(scale_hbm, scale_v)
    scale16 = scale_v[...]

    def body(it, _):
        pltpu.sync_copy(row_hbm.at[pl.ds(base + it * CH, CH)], idx_r)
        pltpu.sync_copy(col_hbm.at[pl.ds(base + it * CH, CH)], idx_c)
        for j in range(CH // 16):
            r16 = idx_r[pl.ds(j * 16, 16)]
            c16 = idx_c[pl.ds(j * 16, 16)]
            dr = plsc.load_gather(dinv_v, [r16])
            dc = plsc.load_gather(dinv_v, [c16])
            w = jnp.where(r16 == c16, 0.0, 1.0).astype(jnp.float32)
            lw_v[pl.ds(j * 16, 16)] = scale16 * dr * dc * w
        pltpu.sync_copy(lw_v, out_hbm.at[pl.ds(base + it * CH, CH)])
        return ()

    lax.fori_loop(0, NCHUNK, body, (), unroll=False)


def _lap_weights(row, col, dinv, scale16):
    kfn = pl.kernel(
        _lapw_kernel,
        out_type=jax.ShapeDtypeStruct((E,), jnp.float32),
        mesh=_mesh(),
        compiler_params=pltpu.CompilerParams(needs_layout_passes=False),
        scratch_types=[
            pltpu.VMEM((NPAD,), jnp.float32),
            pltpu.VMEM((CH,), jnp.int32),
            pltpu.VMEM((CH,), jnp.int32),
            pltpu.VMEM((CH,), jnp.float32),
            pltpu.VMEM((16,), jnp.float32),
        ],
    )
    return kfn(row, col, dinv, scale16)


def _matvec_kernel(F, v_hbm, row3_hbm, col_hbm, lw_hbm, zeros_hbm, out_hbm,
                   idxr_all, idxc_all, lw_all, rows2, sem0, sem1,
                   semp0, semp1, acc_sh):
    c = lax.axis_index("c")
    s = lax.axis_index("s")
    wid = _wid(c, s)
    base = wid * EPW
    FG = F // 16

    h_r = pltpu.async_copy(row3_hbm.at[wid], idxr_all, semp0)
    h_c = pltpu.async_copy(col_hbm.at[pl.ds(base, EPW)], idxc_all, semp1)
    h_l = pltpu.async_copy(lw_hbm.at[pl.ds(base, EPW)], lw_all, sem1)

    @pl.when(s < NS - 1)
    def _():
        cp = pltpu.make_async_copy(zeros_hbm.at[pl.ds(0, 624)],
                                   acc_sh.at[pl.ds(s * 624, 624)], sem0)
        cp.start()
        cp.wait()

    @pl.when(s == NS - 1)
    def _():
        cp = pltpu.make_async_copy(zeros_hbm,
                                   acc_sh.at[pl.ds((NS - 1) * 624, 640)],
                                   sem0)
        cp.start()
        cp.wait()

    h_r.wait()
    h_c.wait()
    h_l.wait()
    plsc.subcore_barrier()

    def start_gather(ck, buf, sem):
        return pltpu.async_copy(
            v_hbm.at[idxc_all.at[pl.ds(ck * CH, CH)]], buf, sem)

    def drain(buf, sem):
        pltpu.make_async_copy(v_hbm.at[pl.ds(0, CH)], buf, sem).wait()

    def scale_scatter(ck, buf):
        @plsc.parallel_loop(0, CH // 8, 1, unroll=2)
        def _(i8):
            for d in range(8):
                e = i8 * 8 + d
                lwb = plsc.load_gather(
                    lw_all, [jnp.full((16,), ck * CH + e, jnp.int32)])
                for j in range(FG):
                    buf[e, pl.ds(j * 16, 16)] = buf[e, pl.ds(j * 16, 16)] * lwb
        pltpu.sync_copy(buf, acc_sh.at[idxr_all.at[ck]], add=True)

    start_gather(0, rows2.at[0], sem0)

    def pair_body(i, _):
        c0 = 2 * i
        h1 = start_gather(c0 + 1, rows2.at[1], sem1)
        drain(rows2.at[0], sem0)
        scale_scatter(c0, rows2.at[0])
        start_gather(c0 + 2, rows2.at[0], sem0)
        h1.wait()
        scale_scatter(c0 + 1, rows2.at[1])
        return ()

    lax.fori_loop(0, (NCHUNK - 1) // 2, pair_body, (), unroll=False)
    drain(rows2.at[0], sem0)
    scale_scatter(NCHUNK - 1, rows2.at[0])
    plsc.subcore_barrier()

    @pl.when(s < NS - 1)
    def _():
        pltpu.sync_copy(acc_sh.at[pl.ds(s * 624, 624)],
                        out_hbm.at[c, pl.ds(s * 624, 624)])

    @pl.when(s == NS - 1)
    def _():
        pltpu.sync_copy(acc_sh.at[pl.ds((NS - 1) * 624, 640)],
                        out_hbm.at[c, pl.ds((NS - 1) * 624, 640)])


def _sc_matvec(v, row3, col, lw, zeros_tile, F):
    kfn = pl.kernel(
        functools.partial(_matvec_kernel, F),
        out_type=jax.ShapeDtypeStruct((NC, N, F), jnp.float32),
        mesh=_mesh(),
        compiler_params=pltpu.CompilerParams(needs_layout_passes=False,
                                             use_tc_tiling_on_sc=False),
        scratch_types=[
            pltpu.VMEM((NCHUNK, CH), jnp.int32),
            pltpu.VMEM((EPW,), jnp.int32),
            pltpu.VMEM((EPW,), jnp.float32),
            pltpu.VMEM((2, CH, F), jnp.float32),
            pltpu.SemaphoreType.DMA,
            pltpu.SemaphoreType.DMA,
            pltpu.SemaphoreType.DMA,
            pltpu.SemaphoreType.DMA,
            pltpu.VMEM_SHARED((N, F), jnp.float32),
        ],
    )
    return kfn(v, row3, col, lw, zeros_tile)


BLK = 1000
GRID = N // BLK


def _step1_body(diag_ref, p_ref, v_ref, w0_ref, w1_ref, t1_ref, out_ref):
    diag = diag_ref[0, 0]
    v = v_ref[...]
    t1 = p_ref[0] + p_ref[1] + diag * v
    t1_ref[...] = t1
    out_ref[...] = (jnp.dot(v, w0_ref[...], preferred_element_type=jnp.float32)
                    + jnp.dot(t1, w1_ref[...], preferred_element_type=jnp.float32))


def _tc_step1(p, v, w0, w1, diag, F):
    return pl.pallas_call(
        _step1_body,
        grid=(GRID,),
        in_specs=[
            pl.BlockSpec((1, 1), lambda i: (0, 0), memory_space=pltpu.SMEM),
            pl.BlockSpec((NC, BLK, F), lambda i: (0, i, 0)),
            pl.BlockSpec((BLK, F), lambda i: (i, 0)),
            pl.BlockSpec((F, H), lambda i: (0, 0)),
            pl.BlockSpec((F, H), lambda i: (0, 0)),
        ],
        out_specs=[
            pl.BlockSpec((BLK, F), lambda i: (i, 0)),
            pl.BlockSpec((BLK, H), lambda i: (i, 0)),
        ],
        out_shape=[
            jax.ShapeDtypeStruct((N, F), jnp.float32),
            jax.ShapeDtypeStruct((N, H), jnp.float32),
        ],
    )(diag, p, v, w0, w1)


def _stepk_body(last, diag_ref, p_ref, v_ref, tp_ref, wk_ref, oprev_ref,
                b_ref, tn_ref, out_ref):
    diag = diag_ref[0, 0]
    tn = 2.0 * (p_ref[0] + p_ref[1] + diag * v_ref[...]) - tp_ref[...]
    tn_ref[...] = tn
    o = oprev_ref[...] + jnp.dot(tn, wk_ref[...],
                                 preferred_element_type=jnp.float32)
    if last:
        o = jnp.maximum(o + b_ref[...], 0.0)
    out_ref[...] = o


def _tc_stepk(p, v, tprev, wk, outprev, b, diag, F, last):
    return pl.pallas_call(
        functools.partial(_stepk_body, last),
        grid=(GRID,),
        in_specs=[
            pl.BlockSpec((1, 1), lambda i: (0, 0), memory_space=pltpu.SMEM),
            pl.BlockSpec((NC, BLK, F), lambda i: (0, i, 0)),
            pl.BlockSpec((BLK, F), lambda i: (i, 0)),
            pl.BlockSpec((BLK, F), lambda i: (i, 0)),
            pl.BlockSpec((F, H), lambda i: (0, 0)),
            pl.BlockSpec((BLK, H), lambda i: (i, 0)),
            pl.BlockSpec((1, H), lambda i: (0, 0)),
        ],
        out_specs=[
            pl.BlockSpec((BLK, F), lambda i: (i, 0)),
            pl.BlockSpec((BLK, H), lambda i: (i, 0)),
        ],
        out_shape=[
            jax.ShapeDtypeStruct((N, F), jnp.float32),
            jax.ShapeDtypeStruct((N, H), jnp.float32),
        ],
    )(diag, p, v, tprev, wk, outprev, b)


def _pool_body(h_ref, f1w_ref, f1b_ref, f2w_ref, f2b_ref, out_ref, acc_ref):
    i = pl.program_id(0)

    @pl.when(i == 0)
    def _():
        acc_ref[...] = jnp.zeros_like(acc_ref)

    acc_ref[...] += jnp.sum(h_ref[...], axis=0, keepdims=True)

    @pl.when(i == GRID - 1)
    def _():
        z = jnp.maximum(
            jnp.dot(acc_ref[...], f1w_ref[...],
                    preferred_element_type=jnp.float32) + f1b_ref[...], 0.0)
        out_ref[...] = (jnp.dot(z, f2w_ref[...],
                                preferred_element_type=jnp.float32)
                        + f2b_ref[...])


def _tc_pool(h, f1w, f1b, f2w, f2b):
    return pl.pallas_call(
        _pool_body,
        grid=(GRID,),
        in_specs=[
            pl.BlockSpec((BLK, H), lambda i: (i, 0)),
            pl.BlockSpec((H, 32), lambda i: (0, 0)),
            pl.BlockSpec((1, 32), lambda i: (0, 0)),
            pl.BlockSpec((32, 1), lambda i: (0, 0)),
            pl.BlockSpec((1, 1), lambda i: (0, 0)),
        ],
        out_specs=pl.BlockSpec((1, 1), lambda i: (0, 0)),
        out_shape=jax.ShapeDtypeStruct((1, 1), jnp.float32),
        scratch_shapes=[pltpu.VMEM((1, H), jnp.float32)],
    )(h, f1w, f1b, f2w, f2b)


def _cheb_layer(v, W, b, row3, col, lw, diag, zeros_tile):
    F = v.shape[1]
    p = _sc_matvec(v, row3, col, lw, zeros_tile, F)
    t1, out = _tc_step1(p, v, W[0], W[1], diag, F)
    t0 = v
    for k in range(2, K):
        p = _sc_matvec(t1, row3, col, lw, zeros_tile, F)
        tn, out = _tc_stepk(p, t1, t0, W[k], out, b, diag, F, last=(k == K - 1))
        t0, t1 = t1, tn
    return out


def kernel(x, edge_index, lmax, batch, W1, b1, W2, b2, W3, b3, W4, b4,
           W5, b5, fc1_W, fc1_b, fc2_W, fc2_b):
    row = edge_index[0]
    col = edge_index[1]
    lm = lmax[0]
    scale16 = jnp.full((16,), -2.0 / lm, jnp.float32)
    diag = jnp.reshape(2.0 / lm - 1.0, (1, 1)).astype(jnp.float32)

    degp = _deg_partials(row, col)
    dinv = _tc_dinv(degp)
    lw = _lap_weights(row, col, dinv, scale16)

    zeros_d = jnp.zeros((640, D), jnp.float32)
    zeros_h = jnp.zeros((640, H), jnp.float32)

    row3 = jnp.reshape(row, (NW, NCHUNK, CH))

    h = _cheb_layer(x, W1, jnp.reshape(b1, (1, H)), row3, col, lw, diag, zeros_d)
    for W, b in ((W2, b2), (W3, b3), (W4, b4), (W5, b5)):
        h = _cheb_layer(h, W, jnp.reshape(b, (1, H)), row3, col, lw, diag, zeros_h)

    return _tc_pool(h, fc1_W, jnp.reshape(fc1_b, (1, 32)), fc2_W,
                    jnp.reshape(fc2_b, (1, 1)))
